# baseline (speedup 1.0000x reference)
import functools

import jax
import jax.numpy as jnp
from jax import lax
from jax.experimental import pallas as pl
from jax.experimental.pallas import tpu as pltpu
from jax.experimental.pallas import tpu_sc as plsc

_info = plsc.get_sparse_core_info()
_NC = _info.num_cores
_NS = _info.num_subcores
_NW = _NC * _NS

_CHUNK = 128


@functools.cache
def _build_gather(tot, d):
    assert tot % (_NW * _CHUNK) == 0
    n_chunks = (tot // _NW) // _CHUNK
    b_per_w = n_chunks * _CHUNK

    mesh = plsc.VectorSubcoreMesh(core_axis_name="c", subcore_axis_name="s")

    @functools.partial(
        pl.kernel,
        out_type=jax.ShapeDtypeStruct((tot, d), jnp.float32),
        mesh=mesh,
        scratch_types=[
            pltpu.VMEM((n_chunks, _CHUNK), jnp.int32),
            pltpu.VMEM((b_per_w, d), jnp.float32),
        ]
        + [pltpu.SemaphoreType.DMA] * (n_chunks + 1),
    )
    def emb_kernel(idx_hbm, table_hbm, out_hbm, idx_v, rows_v, *sems):
        sem_s = sems[n_chunks]
        wid = lax.axis_index("s") * _NC + lax.axis_index("c")
        pltpu.sync_copy(idx_hbm.at[wid], idx_v)
        gathers = []
        for j in range(n_chunks):
            gathers.append(
                pltpu.async_copy(
                    table_hbm.at[idx_v.at[j]],
                    rows_v.at[pl.ds(j * _CHUNK, _CHUNK)],
                    sems[j],
                )
            )
        scatters = []
        for j in range(n_chunks):
            gathers[j].wait()
            scatters.append(
                pltpu.async_copy(
                    rows_v.at[pl.ds(j * _CHUNK, _CHUNK)],
                    out_hbm.at[pl.ds(wid * b_per_w + j * _CHUNK, _CHUNK)],
                    sem_s,
                )
            )
        for cp in scatters:
            cp.wait()

    return emb_kernel


def kernel(x, W):
    b, p = x.shape
    d = W.shape[1]
    tot = b * p
    idx = x.T.reshape(_NW, (tot // _NW) // _CHUNK, _CHUNK).astype(jnp.int32)
    out_t = _build_gather(tot, d)(idx, W)
    return out_t.reshape(p, b, d).transpose(1, 0, 2)


# speedup vs baseline: 1.2503x; 1.2503x over previous
import functools

import jax
import jax.numpy as jnp
from jax import lax
from jax.experimental import pallas as pl
from jax.experimental.pallas import tpu as pltpu
from jax.experimental.pallas import tpu_sc as plsc

_info = plsc.get_sparse_core_info()
_NC = _info.num_cores
_NS = _info.num_subcores
_NW = _NC * _NS

_CHUNK = 128


@functools.cache
def _build_gather(tot, v, d):
    assert tot % (_NW * _CHUNK) == 0
    n_chunks = (tot // _NW) // _CHUNK
    b_per_w = n_chunks * _CHUNK
    rows_per_tile = (-(-v // _NS) + 7) & ~7
    full_tiles = min(_NS - 1, v // rows_per_tile)
    rows_rem = v - full_tiles * rows_per_tile

    mesh = plsc.VectorSubcoreMesh(core_axis_name="c", subcore_axis_name="s")

    @functools.partial(
        pl.kernel,
        out_type=jax.ShapeDtypeStruct((tot, d), jnp.float32),
        mesh=mesh,
        scratch_types=[
            pltpu.VMEM((n_chunks, _CHUNK), jnp.int32),
            pltpu.VMEM((b_per_w, d), jnp.float32),
            pltpu.VMEM_SHARED((v, d), jnp.float32),
        ]
        + [pltpu.SemaphoreType.DMA] * (n_chunks + 1),
    )
    def emb_kernel(idx_hbm, table_hbm, out_hbm, idx_v, rows_v, table_sp,
                   *sems):
        sem_s = sems[n_chunks]
        sid = lax.axis_index("s")
        wid = sid * _NC + lax.axis_index("c")
        @pl.when(sid < full_tiles)
        def _():
            pltpu.sync_copy(
                table_hbm.at[pl.ds(sid * rows_per_tile, rows_per_tile)],
                table_sp.at[pl.ds(sid * rows_per_tile, rows_per_tile)],
            )
        if rows_rem:
            @pl.when(sid == full_tiles)
            def _():
                pltpu.sync_copy(
                    table_hbm.at[pl.ds(full_tiles * rows_per_tile, rows_rem)],
                    table_sp.at[pl.ds(full_tiles * rows_per_tile, rows_rem)],
                )
        pltpu.sync_copy(idx_hbm.at[wid], idx_v)
        plsc.subcore_barrier()
        gathers = []
        for j in range(n_chunks):
            gathers.append(
                pltpu.async_copy(
                    table_sp.at[idx_v.at[j]],
                    rows_v.at[pl.ds(j * _CHUNK, _CHUNK)],
                    sems[j],
                )
            )
        scatters = []
        for j in range(n_chunks):
            gathers[j].wait()
            scatters.append(
                pltpu.async_copy(
                    rows_v.at[pl.ds(j * _CHUNK, _CHUNK)],
                    out_hbm.at[pl.ds(wid * b_per_w + j * _CHUNK, _CHUNK)],
                    sem_s,
                )
            )
        for cp in scatters:
            cp.wait()

    return emb_kernel


def kernel(x, W):
    b, p = x.shape
    v, d = W.shape
    tot = b * p
    idx = x.T.reshape(_NW, (tot // _NW) // _CHUNK, _CHUNK).astype(jnp.int32)
    out_t = _build_gather(tot, v, d)(idx, W)
    return out_t.reshape(p, b, d).transpose(1, 0, 2)
